# baseline (device time: 41014 ns/iter reference)
import jax
import jax.numpy as jnp
from jax import lax
from jax.experimental import pallas as pl
from jax.experimental.pallas import tpu as pltpu

NH = 8


def kernel(Q, K, V):
    b, s, h, d = Q.shape
    scale = d ** -0.5

    def body(q_ref, k_ref, v_ref, out_ref, kv_send, kv_recv, o_send, o_recv,
             kv_ssem, kv_rsem, o_ssem, o_rsem):
        my_x = lax.axis_index("x")
        my_y = lax.axis_index("y")
        y_nbr = (my_x, 1 - my_y)
        x_nbr = (1 - my_x, my_y)
        bm = my_x
        bo = 1 - my_x

        barrier = pltpu.get_barrier_semaphore()
        for nbr in (y_nbr, x_nbr):
            pl.semaphore_signal(
                barrier, inc=1, device_id=nbr,
                device_id_type=pl.DeviceIdType.MESH,
            )
        pl.semaphore_wait(barrier, 2)

        k_bf = k_ref[bm].astype(jnp.bfloat16)
        v_bf = v_ref[bm].astype(jnp.bfloat16)
        kv_rdmas = []
        for c in range(NH):
            kv_send[c, 0] = k_bf[:, c, :]
            kv_send[c, 1] = v_bf[:, c, :]
            r = pltpu.make_async_remote_copy(
                src_ref=kv_send.at[c], dst_ref=kv_recv.at[c],
                send_sem=kv_ssem.at[c], recv_sem=kv_rsem.at[c],
                device_id=y_nbr, device_id_type=pl.DeviceIdType.MESH,
            )
            r.start()
            kv_rdmas.append(r)

        q_bf = q_ref[bm].astype(jnp.bfloat16)

        o_rdmas = []
        for c in range(NH):
            kv_rdmas[c].wait_recv()
            k_all = jnp.concatenate([kv_send[c, 0], kv_recv[c, 0]], axis=0)
            v_all = jnp.concatenate([kv_send[c, 1], kv_recv[c, 1]], axis=0)
            s_i = lax.dot_general(
                q_bf[:, c, :], k_all, (((1,), (1,)), ((), ())),
                preferred_element_type=jnp.float32,
            ) * scale
            p = jnp.exp(s_i)
            l = jnp.sum(p, axis=1, keepdims=True)
            o_i = lax.dot_general(
                p.astype(jnp.bfloat16), v_all, (((1,), (0,)), ((), ())),
                preferred_element_type=jnp.float32,
            ) / l
            out_ref[bm, :, c, :] = o_i
            o_send[c] = o_i.astype(jnp.bfloat16)
            r = pltpu.make_async_remote_copy(
                src_ref=o_send.at[c], dst_ref=o_recv.at[c],
                send_sem=o_ssem.at[c], recv_sem=o_rsem.at[c],
                device_id=x_nbr, device_id_type=pl.DeviceIdType.MESH,
            )
            r.start()
            o_rdmas.append(r)
            if c > 0:
                o_rdmas[c - 1].wait_recv()
                out_ref[bo, :, c - 1, :] = o_recv[c - 1].astype(jnp.float32)

        o_rdmas[NH - 1].wait_recv()
        out_ref[bo, :, NH - 1, :] = o_recv[NH - 1].astype(jnp.float32)

        for c in range(NH):
            kv_rdmas[c].wait_send()
            o_rdmas[c].wait_send()

    return pl.pallas_call(
        body,
        out_shape=jax.ShapeDtypeStruct((b, s, h, d), jnp.float32),
        in_specs=[pl.BlockSpec(memory_space=pltpu.VMEM)] * 3,
        out_specs=pl.BlockSpec(memory_space=pltpu.VMEM),
        scratch_shapes=[
            pltpu.VMEM((NH, 2, s, d), jnp.bfloat16),
            pltpu.VMEM((NH, 2, s, d), jnp.bfloat16),
            pltpu.VMEM((NH, s, d), jnp.bfloat16),
            pltpu.VMEM((NH, s, d), jnp.bfloat16),
            pltpu.SemaphoreType.DMA((NH,)),
            pltpu.SemaphoreType.DMA((NH,)),
            pltpu.SemaphoreType.DMA((NH,)),
            pltpu.SemaphoreType.DMA((NH,)),
        ],
        compiler_params=pltpu.CompilerParams(collective_id=0),
    )(Q, K, V)
